# Initial kernel scaffold; baseline (speedup 1.0000x reference)
import jax
import jax.numpy as jnp
from jax.experimental import pallas as pl


def kernel(x, edge_index, W1, b1, W2, b2, a1_init, a1_root, a1_bias, a2_init, a2_root, a2_bias, a3_init, a3_root, a3_bias, a4_init, a4_root, a4_bias, W3, b3, W4, b4):
    raise NotImplementedError("write your pallas kernel here")


# speedup vs baseline: 74.2744x; 74.2744x over previous
import functools

import jax
import jax.numpy as jnp
from jax import lax
from jax.experimental import pallas as pl
from jax.experimental.pallas import tpu as pltpu
from jax.experimental.pallas import tpu_sc as plsc

N = 50000
E = 800000
F = 32
NC = 2
NS = 16
NW = NC * NS
C = 128
NCH = E // C
DW = 8

SROWS = 3128
ZCH = 616
ZREP = 5
TAIL = SROWS - ZREP * ZCH

@functools.cache
def _mesh():
    return plsc.VectorSubcoreMesh(
        core_axis_name="c", subcore_axis_name="s",
        num_cores=NC, num_subcores=NS)


def _zero_vmem(buf, width):
    z16 = jnp.zeros((16,), jnp.float32)

    def zb(i, carry):
        for j in range(width // 16):
            buf[i, pl.ds(j * 16, 16)] = z16
        return carry

    lax.fori_loop(0, buf.shape[0], zb, None)


def _sweep(s, fn):
    lo = s * SROWS
    for j in range(ZREP):
        fn(lo + j * ZCH, ZCH)

    @pl.when(s < NS - 1)
    def _():
        fn(lo + ZREP * ZCH, TAIL)


def _prop_body(src, rowi, coli, out, idx_r, idx_c, rows_v, zbuf, accum, sem):
    c = lax.axis_index("c")
    s = lax.axis_index("s")
    w = s * NC + c

    _zero_vmem(zbuf, F)
    _sweep(s, lambda off, n: pltpu.sync_copy(
        zbuf.at[pl.ds(0, n)], accum.at[pl.ds(off, n)]))
    plsc.subcore_barrier()

    nw = (NCH - 1 - w) // NW + 1

    def step(t, carry):
        base = (w + NW * t) * C
        pltpu.sync_copy(rowi.at[pl.ds(base, C)], idx_r)
        pltpu.sync_copy(coli.at[pl.ds(base, C)], idx_c.at[0])
        pltpu.async_copy(src.at[idx_r], rows_v, sem).wait()
        pltpu.sync_copy(rows_v, accum.at[idx_c.at[0]], add=True)
        return carry

    lax.fori_loop(0, nw, step, None)
    plsc.subcore_barrier()

    _sweep(s, lambda off, n: pltpu.sync_copy(
        accum.at[pl.ds(off, n)], out.at[c, pl.ds(off, n)]))


@functools.cache
def _propagate_kernel():
    return pl.kernel(
        _prop_body,
        out_type=jax.ShapeDtypeStruct((NC, N, F), jnp.float32),
        mesh=_mesh(),
        compiler_params=pltpu.CompilerParams(use_tc_tiling_on_sc=False),
        scratch_types=[
            pltpu.VMEM((C,), jnp.int32),
            pltpu.VMEM((1, C), jnp.int32),
            pltpu.VMEM((C, F), jnp.float32),
            pltpu.VMEM((ZCH, F), jnp.float32),
            pltpu.VMEM_SHARED((N, F), jnp.float32),
            pltpu.SemaphoreType.DMA,
        ],
    )


def _propagate(src, row, col):
    return _propagate_kernel()(src, row, col)


def _deg_body(coli, ones_c, zer_c, out, idx_c, ones_v, zbuf, accum):
    c = lax.axis_index("c")
    s = lax.axis_index("s")
    w = s * NC + c

    pltpu.sync_copy(ones_c, ones_v)
    pltpu.sync_copy(zer_c, zbuf)
    _sweep(s, lambda off, n: pltpu.sync_copy(
        zbuf.at[pl.ds(0, n)], accum.at[pl.ds(off, n)]))
    plsc.subcore_barrier()

    nw = (NCH - 1 - w) // NW + 1

    def step(t, carry):
        base = (w + NW * t) * C
        pltpu.sync_copy(coli.at[pl.ds(base, C)], idx_c.at[0])
        pltpu.sync_copy(ones_v, accum.at[idx_c.at[0]], add=True)
        return carry

    lax.fori_loop(0, nw, step, None)
    plsc.subcore_barrier()

    _sweep(s, lambda off, n: pltpu.sync_copy(
        accum.at[pl.ds(off, n)], out.at[c, pl.ds(off, n)]))


@functools.cache
def _degree_kernel():
    return pl.kernel(
        _deg_body,
        out_type=jax.ShapeDtypeStruct((NC, N, DW), jnp.float32),
        mesh=_mesh(),
        compiler_params=pltpu.CompilerParams(use_tc_tiling_on_sc=False),
        scratch_types=[
            pltpu.VMEM((1, C), jnp.int32),
            pltpu.VMEM((C, DW), jnp.float32),
            pltpu.VMEM((ZCH, DW), jnp.float32),
            pltpu.VMEM_SHARED((N, DW), jnp.float32),
        ],
    )


def _degree(col):
    ones_c = jnp.zeros((C, DW), jnp.float32).at[:, 0].set(1.0)
    zer_c = jnp.zeros((ZCH, DW), jnp.float32)
    return _degree_kernel()(col, ones_c, zer_c)


R = 1000
G = N // R

def _dot(a, b):
    return jnp.dot(a, b, precision="highest")


def _full(shape):
    nd = len(shape)
    return pl.BlockSpec(shape, lambda i, _nd=nd: (0,) * _nd)


def _rows(shape, axis):
    def idx(i, _axis=axis, _nd=len(shape)):
        return tuple(i if d == _axis else 0 for d in range(_nd))
    return pl.BlockSpec(shape, idx)


def _tc1_body(x_r, degp_r, w1_r, b1_r, w2_r, b2_r, ai_r, ar_r,
              src_r, root_r, dis_r):
    deg = degp_r[0] + degp_r[1]
    disf = jnp.where(deg > 0, lax.rsqrt(jnp.maximum(deg, 1e-12)), 0.0)
    dis = disf[:, 0:1]
    dis_r[...] = dis
    h1 = jnp.maximum(_dot(x_r[...], w1_r[...]) + b1_r[...], 0.0)
    h = jnp.maximum(_dot(h1, w2_r[...]) + b2_r[...], 0.0)
    hs = dis * h
    for k in range(4):
        src_r[k] = _dot(hs, ai_r[k])
        root_r[k] = _dot(h, ar_r[k])


def _tc1(x, degp, W1, b1, W2, b2, A1i, A1r):
    return pl.pallas_call(
        _tc1_body,
        grid=(G,),
        in_specs=[
            _rows((R, 4), 0),
            _rows((NC, R, DW), 1),
            _full((4, 32)), _full((1, 32)),
            _full((32, 128)), _full((1, 128)),
            _full((4, 128, 32)), _full((4, 128, 32)),
        ],
        out_specs=[
            _rows((4, R, 32), 1),
            _rows((4, R, 32), 1),
            _rows((R, 1), 0),
        ],
        out_shape=[
            jax.ShapeDtypeStruct((4, N, 32), jnp.float32),
            jax.ShapeDtypeStruct((4, N, 32), jnp.float32),
            jax.ShapeDtypeStruct((N, 1), jnp.float32),
        ],
    )(x, degp, W1, b1, W2, b2, A1i, A1r)


def _tc2_body(p0_r, p1_r, p2_r, p3_r, root_r, bias_r, dis_r, h_r, hs_r):
    dis = dis_r[...]
    acc = jnp.zeros((R, 32), jnp.float32)
    for k, p_r in enumerate((p0_r, p1_r, p2_r, p3_r)):
        agg = dis * (p_r[0] + p_r[1])
        acc = acc + jnp.maximum(agg + root_r[k] + bias_r[k], 0.0)
    h = acc * 0.25
    h_r[...] = h
    hs_r[...] = dis * h


def _tc2(p0, p1, p2, p3, root1, bias1, dis):
    return pl.pallas_call(
        _tc2_body,
        grid=(G,),
        in_specs=[
            _rows((NC, R, 32), 1), _rows((NC, R, 32), 1),
            _rows((NC, R, 32), 1), _rows((NC, R, 32), 1),
            _rows((4, R, 32), 1),
            _full((4, 32)),
            _rows((R, 1), 0),
        ],
        out_specs=[_rows((R, 32), 0), _rows((R, 32), 0)],
        out_shape=[
            jax.ShapeDtypeStruct((N, 32), jnp.float32),
            jax.ShapeDtypeStruct((N, 32), jnp.float32),
        ],
    )(p0, p1, p2, p3, root1, bias1, dis)


def _tc3_body(p_r, hprev_r, dis_r, ai_r, ar_r, bias_r, h_r, hs_r):
    dis = dis_r[...]
    ps = dis * (p_r[0] + p_r[1])
    hp = hprev_r[...]
    acc = jnp.zeros((R, 32), jnp.float32)
    for k in range(4):
        acc = acc + jnp.maximum(
            _dot(ps, ai_r[k]) + _dot(hp, ar_r[k]) + bias_r[k], 0.0)
    h = acc * 0.25
    h_r[...] = h
    hs_r[...] = dis * h


def _tc3(p, hprev, dis, Ai, Ar, bias):
    return pl.pallas_call(
        _tc3_body,
        grid=(G,),
        in_specs=[
            _rows((NC, R, 32), 1),
            _rows((R, 32), 0),
            _rows((R, 1), 0),
            _full((4, 32, 32)), _full((4, 32, 32)), _full((4, 32)),
        ],
        out_specs=[_rows((R, 32), 0), _rows((R, 32), 0)],
        out_shape=[
            jax.ShapeDtypeStruct((N, 32), jnp.float32),
            jax.ShapeDtypeStruct((N, 32), jnp.float32),
        ],
    )(p, hprev, dis, Ai, Ar, bias)


def _tc4_body(h_r, w3_r, b3_r, w4_r, b4_r, out_r):
    t = jnp.maximum(_dot(h_r[...], w3_r[...]) + b3_r[...], 0.0)
    out_r[...] = _dot(t, w4_r[...]) + b4_r[...]


def _tc4(h, W3, b3, W4, b4):
    return pl.pallas_call(
        _tc4_body,
        grid=(G,),
        in_specs=[
            _rows((R, 32), 0),
            _full((32, 16)), _full((1, 16)),
            _full((16, 2)), _full((1, 2)),
        ],
        out_specs=_rows((R, 2), 0),
        out_shape=jax.ShapeDtypeStruct((N, 2), jnp.float32),
    )(h, W3, b3, W4, b4)


def kernel(x, edge_index, W1, b1, W2, b2,
           a1_init, a1_root, a1_bias, a2_init, a2_root, a2_bias,
           a3_init, a3_root, a3_bias, a4_init, a4_root, a4_bias,
           W3, b3, W4, b4):
    row = edge_index[0]
    col = edge_index[1]

    degp = _degree(col)
    src, root1, dis = _tc1(x, degp, W1, b1.reshape(1, 32),
                           W2, b2.reshape(1, 128), a1_init, a1_root)

    p0 = _propagate(src[0], row, col)
    p1 = _propagate(src[1], row, col)
    p2 = _propagate(src[2], row, col)
    p3 = _propagate(src[3], row, col)
    h, hs = _tc2(p0, p1, p2, p3, root1, a1_bias.reshape(4, 32), dis)

    for Ai, Ar, Ab in ((a2_init, a2_root, a2_bias),
                       (a3_init, a3_root, a3_bias),
                       (a4_init, a4_root, a4_bias)):
        pp = _propagate(hs, row, col)
        h, hs = _tc3(pp, h, dis, Ai, Ar, Ab.reshape(4, 32))

    return _tc4(h, W3, b3.reshape(1, 16), W4, b4.reshape(1, 2))


# speedup vs baseline: 90.9504x; 1.2245x over previous
import functools

import jax
import jax.numpy as jnp
from jax import lax
from jax.experimental import pallas as pl
from jax.experimental.pallas import tpu as pltpu
from jax.experimental.pallas import tpu_sc as plsc

N = 50000
E = 800000
F = 32
NC = 2
NS = 16
NW = NC * NS
C = 128
NCH = E // C
IBLK = 4
NIBF = NCH // IBLK
NTAIL = NCH - NIBF * IBLK

SROWS = 3128
ZCH = 88
ZREP = 35
TAIL = SROWS - ZREP * ZCH

@functools.cache
def _mesh():
    return plsc.VectorSubcoreMesh(
        core_axis_name="c", subcore_axis_name="s",
        num_cores=NC, num_subcores=NS)


def _zero_vmem(buf, width):
    z16 = jnp.zeros((16,), jnp.float32)

    def zb(i, carry):
        for j in range(width // 16):
            buf[i, pl.ds(j * 16, 16)] = z16
        return carry

    lax.fori_loop(0, buf.shape[0], zb, None)


def _sweep(s, fn):
    lo = s * SROWS
    for j in range(ZREP):
        fn(lo + j * ZCH, ZCH)

    @pl.when(s < NS - 1)
    def _():
        fn(lo + ZREP * ZCH, TAIL)


def _prop_pass(src, rowi, coli, out, c, s, w,
               idx_r, idx_c, rows_v, zbuf, accum, gsem, ssem):
    _sweep(s, lambda off, n: pltpu.sync_copy(
        zbuf.at[pl.ds(0, n)], accum.at[pl.ds(off, n)]))
    plsc.subcore_barrier()

    nib = (NIBF - 1 - w) // NW + 1

    def chunks(ebase, m):
        for j in range(m):
            pltpu.sync_copy(coli.at[pl.ds(ebase + j * C, C)], idx_c.at[j])
        gs = [pltpu.async_copy(
            src.at[idx_r.at[pl.ds(j * C, C)]], rows_v.at[j], gsem)
            for j in range(m)]
        ss = []
        for j in range(m):
            gs[j].wait()
            ss.append(pltpu.async_copy(
                rows_v.at[j], accum.at[idx_c.at[j]], ssem, add=True))
        for j in range(m):
            ss[j].wait()

    def iblock(t, carry):
        ebase = (w + NW * t) * (IBLK * C)
        pltpu.sync_copy(rowi.at[pl.ds(ebase, IBLK * C)], idx_r)
        chunks(ebase, IBLK)
        return carry

    lax.fori_loop(0, nib, iblock, None)

    @pl.when(w == NW - 1)
    def _():
        ebase = NIBF * IBLK * C
        pltpu.sync_copy(rowi.at[pl.ds(ebase, NTAIL * C)],
                        idx_r.at[pl.ds(0, NTAIL * C)])
        chunks(ebase, NTAIL)

    plsc.subcore_barrier()

    _sweep(s, lambda off, n: pltpu.sync_copy(
        accum.at[pl.ds(off, n)], out.at[c, pl.ds(off, n)]))


def _prop_body(src, rowi, coli, out, idx_r, idx_c, rows_v, zbuf, accum,
               gsem, ssem):
    c = lax.axis_index("c")
    s = lax.axis_index("s")
    w = s * NC + c
    _zero_vmem(zbuf, F)
    _prop_pass(src, rowi, coli, out, c, s, w,
               idx_r, idx_c, rows_v, zbuf, accum, gsem, ssem)


def _prop4_body(s0, s1, s2, s3, rowi, coli, o0, o1, o2, o3,
                idx_r, idx_c, rows_v, zbuf, accum, gsem, ssem):
    c = lax.axis_index("c")
    s = lax.axis_index("s")
    w = s * NC + c
    _zero_vmem(zbuf, F)
    for src, out in ((s0, o0), (s1, o1), (s2, o2), (s3, o3)):
        _prop_pass(src, rowi, coli, out, c, s, w,
                   idx_r, idx_c, rows_v, zbuf, accum, gsem, ssem)
        plsc.subcore_barrier()


_SCRATCH = None


def _sc_scratch():
    global _SCRATCH
    if _SCRATCH is None:
        _SCRATCH = [
            pltpu.VMEM((IBLK * C,), jnp.int32),
            pltpu.VMEM((IBLK, C), jnp.int32),
            pltpu.VMEM((IBLK, C, F), jnp.float32),
            pltpu.VMEM((ZCH, F), jnp.float32),
            pltpu.VMEM_SHARED((N, F), jnp.float32),
            pltpu.SemaphoreType.DMA,
            pltpu.SemaphoreType.DMA,
        ]
    return _SCRATCH


@functools.cache
def _propagate_kernel():
    return pl.kernel(
        _prop_body,
        out_type=jax.ShapeDtypeStruct((NC, N, F), jnp.float32),
        mesh=_mesh(),
        compiler_params=pltpu.CompilerParams(use_tc_tiling_on_sc=False),
        scratch_types=_sc_scratch(),
    )


@functools.cache
def _propagate4_kernel():
    return pl.kernel(
        _prop4_body,
        out_type=[jax.ShapeDtypeStruct((NC, N, F), jnp.float32)] * 4,
        mesh=_mesh(),
        compiler_params=pltpu.CompilerParams(use_tc_tiling_on_sc=False),
        scratch_types=_sc_scratch(),
    )


def _propagate(src, row, col):
    return _propagate_kernel()(src, row, col)


R = 1000
G = N // R

def _dot(a, b):
    return jnp.dot(a, b, precision="highest")


def _full(shape):
    nd = len(shape)
    return pl.BlockSpec(shape, lambda i, _nd=nd: (0,) * _nd)


def _rows(shape, axis):
    def idx(i, _axis=axis, _nd=len(shape)):
        return tuple(i if d == _axis else 0 for d in range(_nd))
    return pl.BlockSpec(shape, idx)


def _tc1_body(x_r, degp_r, w1_r, b1_r, w2_r, b2_r, ai_r, ar_r,
              src_r, root_r, dis_r):
    deg = degp_r[0] + degp_r[1]
    disf = jnp.where(deg > 0, lax.rsqrt(jnp.maximum(deg, 1e-12)), 0.0)
    dis = disf[:, 0:1]
    dis_r[...] = dis
    h1 = jnp.maximum(_dot(x_r[...], w1_r[...]) + b1_r[...], 0.0)
    h = jnp.maximum(_dot(h1, w2_r[...]) + b2_r[...], 0.0)
    hs = dis * h
    for k in range(4):
        src_r[k] = _dot(hs, ai_r[k])
        root_r[k] = _dot(h, ar_r[k])


def _tc1(x, degp, W1, b1, W2, b2, A1i, A1r):
    return pl.pallas_call(
        _tc1_body,
        grid=(G,),
        in_specs=[
            _rows((R, 4), 0),
            _rows((NC, R, F), 1),
            _full((4, 32)), _full((1, 32)),
            _full((32, 128)), _full((1, 128)),
            _full((4, 128, 32)), _full((4, 128, 32)),
        ],
        out_specs=[
            _rows((4, R, 32), 1),
            _rows((4, R, 32), 1),
            _rows((R, 1), 0),
        ],
        out_shape=[
            jax.ShapeDtypeStruct((4, N, 32), jnp.float32),
            jax.ShapeDtypeStruct((4, N, 32), jnp.float32),
            jax.ShapeDtypeStruct((N, 1), jnp.float32),
        ],
    )(x, degp, W1, b1, W2, b2, A1i, A1r)


def _tc2_body(p0_r, p1_r, p2_r, p3_r, root_r, bias_r, dis_r, h_r, hs_r):
    dis = dis_r[...]
    acc = jnp.zeros((R, 32), jnp.float32)
    for k, p_r in enumerate((p0_r, p1_r, p2_r, p3_r)):
        agg = dis * (p_r[0] + p_r[1])
        acc = acc + jnp.maximum(agg + root_r[k] + bias_r[k], 0.0)
    h = acc * 0.25
    h_r[...] = h
    hs_r[...] = dis * h


def _tc2(p0, p1, p2, p3, root1, bias1, dis):
    return pl.pallas_call(
        _tc2_body,
        grid=(G,),
        in_specs=[
            _rows((NC, R, 32), 1), _rows((NC, R, 32), 1),
            _rows((NC, R, 32), 1), _rows((NC, R, 32), 1),
            _rows((4, R, 32), 1),
            _full((4, 32)),
            _rows((R, 1), 0),
        ],
        out_specs=[_rows((R, 32), 0), _rows((R, 32), 0)],
        out_shape=[
            jax.ShapeDtypeStruct((N, 32), jnp.float32),
            jax.ShapeDtypeStruct((N, 32), jnp.float32),
        ],
    )(p0, p1, p2, p3, root1, bias1, dis)


def _tc3_body(p_r, hprev_r, dis_r, ai_r, ar_r, bias_r, h_r, hs_r):
    dis = dis_r[...]
    ps = dis * (p_r[0] + p_r[1])
    hp = hprev_r[...]
    acc = jnp.zeros((R, 32), jnp.float32)
    for k in range(4):
        acc = acc + jnp.maximum(
            _dot(ps, ai_r[k]) + _dot(hp, ar_r[k]) + bias_r[k], 0.0)
    h = acc * 0.25
    h_r[...] = h
    hs_r[...] = dis * h


def _tc3(p, hprev, dis, Ai, Ar, bias):
    return pl.pallas_call(
        _tc3_body,
        grid=(G,),
        in_specs=[
            _rows((NC, R, 32), 1),
            _rows((R, 32), 0),
            _rows((R, 1), 0),
            _full((4, 32, 32)), _full((4, 32, 32)), _full((4, 32)),
        ],
        out_specs=[_rows((R, 32), 0), _rows((R, 32), 0)],
        out_shape=[
            jax.ShapeDtypeStruct((N, 32), jnp.float32),
            jax.ShapeDtypeStruct((N, 32), jnp.float32),
        ],
    )(p, hprev, dis, Ai, Ar, bias)


def _tc4_body(h_r, w3_r, b3_r, w4_r, b4_r, out_r):
    t = jnp.maximum(_dot(h_r[...], w3_r[...]) + b3_r[...], 0.0)
    out_r[...] = _dot(t, w4_r[...]) + b4_r[...]


def _tc4(h, W3, b3, W4, b4):
    return pl.pallas_call(
        _tc4_body,
        grid=(G,),
        in_specs=[
            _rows((R, 32), 0),
            _full((32, 16)), _full((1, 16)),
            _full((16, 2)), _full((1, 2)),
        ],
        out_specs=_rows((R, 2), 0),
        out_shape=jax.ShapeDtypeStruct((N, 2), jnp.float32),
    )(h, W3, b3, W4, b4)


def kernel(x, edge_index, W1, b1, W2, b2,
           a1_init, a1_root, a1_bias, a2_init, a2_root, a2_bias,
           a3_init, a3_root, a3_bias, a4_init, a4_root, a4_bias,
           W3, b3, W4, b4):
    row2 = edge_index[0]
    col2 = edge_index[1]

    e_src = jnp.zeros((N, F), jnp.float32).at[:, 0].set(1.0)
    degp = _propagate(e_src, row2, col2)
    src, root1, dis = _tc1(x, degp, W1, b1.reshape(1, 32),
                           W2, b2.reshape(1, 128), a1_init, a1_root)

    p0, p1, p2, p3 = _propagate4_kernel()(
        src[0], src[1], src[2], src[3], row2, col2)
    h, hs = _tc2(p0, p1, p2, p3, root1, a1_bias.reshape(4, 32), dis)

    for Ai, Ar, Ab in ((a2_init, a2_root, a2_bias),
                       (a3_init, a3_root, a3_bias),
                       (a4_init, a4_root, a4_bias)):
        pp = _propagate(hs, row2, col2)
        h, hs = _tc3(pp, h, dis, Ai, Ar, Ab.reshape(4, 32))

    return _tc4(h, W3, b3.reshape(1, 16), W4, b4.reshape(1, 2))


# speedup vs baseline: 188.3206x; 2.0706x over previous
import functools

import jax
import jax.numpy as jnp
from jax import lax
from jax.experimental import pallas as pl
from jax.experimental.pallas import tpu as pltpu
from jax.experimental.pallas import tpu_sc as plsc

N = 50000
E = 800000
F = 32
NC = 2
NS = 16
NW = NC * NS
C = 128
NCH = E // C
IBLK = 3
NIBF = NCH // IBLK
NTAIL = NCH - NIBF * IBLK
NSEC = 17

SROWS = 3128
ZCH = 88
ZREP = 35
TAIL = SROWS - ZREP * ZCH

@functools.cache
def _mesh():
    return plsc.VectorSubcoreMesh(
        core_axis_name="c", subcore_axis_name="s",
        num_cores=NC, num_subcores=NS)


def _zero_vmem(buf, width):
    z16 = jnp.zeros((16,), jnp.float32)

    def zb(i, carry):
        for j in range(width // 16):
            buf[i, pl.ds(j * 16, 16)] = z16
        return carry

    lax.fori_loop(0, buf.shape[0], zb, None)


def _sweep(s, fn):
    lo = s * SROWS
    for j in range(ZREP):
        fn(lo + j * ZCH, ZCH)

    @pl.when(s < NS - 1)
    def _():
        fn(lo + ZREP * ZCH, TAIL)


def _prop_pass(src, rowi, coli, out, c, s, w,
               idx_r, idx_c, rows_v, zbuf, accum, gsem, ssem, isem):
    _sweep(s, lambda off, n: pltpu.sync_copy(
        zbuf.at[pl.ds(0, n)], accum.at[pl.ds(off, n)]))
    plsc.subcore_barrier()

    nib = (NIBF - 1 - w) // NW + 1

    def fire_idx(b, slot):
        e0 = b * (IBLK * C)
        pltpu.async_copy(rowi.at[pl.ds(e0, IBLK * C)], idx_r.at[slot], isem)
        for j in range(IBLK):
            pltpu.async_copy(coli.at[pl.ds(e0 + j * C, C)],
                             idx_c.at[slot, j], isem)

    def drain_idx(b, slot):
        e0 = b * (IBLK * C)
        pltpu.make_async_copy(rowi.at[pl.ds(e0, IBLK * C)],
                              idx_r.at[slot], isem).wait()
        for j in range(IBLK):
            pltpu.make_async_copy(coli.at[pl.ds(e0 + j * C, C)],
                                  idx_c.at[slot, j], isem).wait()

    def drain_scatters(p):
        for j in range(IBLK):
            pltpu.make_async_copy(src.at[pl.ds(0, C)],
                                  rows_v.at[p, j], ssem).wait()

    for t0 in range(2):
        @pl.when(t0 < nib)
        def _(t0=t0):
            fire_idx(w + NW * t0, t0)

    def section(i, carry):
        for q in range(4):
            p = q % 2

            @pl.when(4 * i + q < nib)
            def _(i=i, q=q, p=p):
                t = 4 * i + q
                b = w + NW * t

                @pl.when(t >= 2)
                def _():
                    drain_scatters(1 - p)

                drain_idx(b, q)
                gs = [pltpu.async_copy(
                    src.at[idx_r.at[q, pl.ds(j * C, C)]],
                    rows_v.at[p, j], gsem) for j in range(IBLK)]

                @pl.when(t + 2 < nib)
                def _():
                    fire_idx(b + 2 * NW, (q + 2) % 4)

                for j in range(IBLK):
                    gs[j].wait()
                    pltpu.async_copy(rows_v.at[p, j],
                                     accum.at[idx_c.at[q, j]], ssem, add=True)
        return carry

    lax.fori_loop(0, NSEC, section, None)

    for p in range(2):
        drain_scatters(p)

    @pl.when(w == NW - 1)
    def _():
        e0 = NIBF * IBLK * C
        pltpu.sync_copy(rowi.at[pl.ds(e0, C)], idx_r.at[0, pl.ds(0, C)])
        pltpu.sync_copy(coli.at[pl.ds(e0, C)], idx_c.at[0, 0])
        pltpu.async_copy(src.at[idx_r.at[0, pl.ds(0, C)]],
                         rows_v.at[0, 0], gsem).wait()
        pltpu.sync_copy(rows_v.at[0, 0], accum.at[idx_c.at[0, 0]], add=True)

    plsc.subcore_barrier()

    _sweep(s, lambda off, n: pltpu.sync_copy(
        accum.at[pl.ds(off, n)], out.at[c, pl.ds(off, n)]))


def _prop_body(src, rowi, coli, out, idx_r, idx_c, rows_v, zbuf, accum,
               gsem, ssem, isem):
    c = lax.axis_index("c")
    s = lax.axis_index("s")
    w = s * NC + c
    _zero_vmem(zbuf, F)
    _prop_pass(src, rowi, coli, out, c, s, w,
               idx_r, idx_c, rows_v, zbuf, accum, gsem, ssem, isem)


def _prop4_body(s0, s1, s2, s3, rowi, coli, o0, o1, o2, o3,
                idx_r, idx_c, rows_v, zbuf, accum, gsem, ssem, isem):
    c = lax.axis_index("c")
    s = lax.axis_index("s")
    w = s * NC + c
    _zero_vmem(zbuf, F)
    for src, out in ((s0, o0), (s1, o1), (s2, o2), (s3, o3)):
        _prop_pass(src, rowi, coli, out, c, s, w,
                   idx_r, idx_c, rows_v, zbuf, accum, gsem, ssem, isem)
        plsc.subcore_barrier()


_SCRATCH = None


def _sc_scratch():
    global _SCRATCH
    if _SCRATCH is None:
        _SCRATCH = [
            pltpu.VMEM((4, IBLK * C), jnp.int32),
            pltpu.VMEM((4, IBLK, C), jnp.int32),
            pltpu.VMEM((2, IBLK, C, F), jnp.float32),
            pltpu.VMEM((ZCH, F), jnp.float32),
            pltpu.VMEM_SHARED((N, F), jnp.float32),
            pltpu.SemaphoreType.DMA,
            pltpu.SemaphoreType.DMA,
            pltpu.SemaphoreType.DMA,
        ]
    return _SCRATCH


@functools.cache
def _propagate_kernel():
    return pl.kernel(
        _prop_body,
        out_type=jax.ShapeDtypeStruct((NC, N, F), jnp.float32),
        mesh=_mesh(),
        compiler_params=pltpu.CompilerParams(use_tc_tiling_on_sc=False),
        scratch_types=_sc_scratch(),
    )


@functools.cache
def _propagate4_kernel():
    return pl.kernel(
        _prop4_body,
        out_type=[jax.ShapeDtypeStruct((NC, N, F), jnp.float32)] * 4,
        mesh=_mesh(),
        compiler_params=pltpu.CompilerParams(use_tc_tiling_on_sc=False),
        scratch_types=_sc_scratch(),
    )


def _propagate(src, row, col):
    return _propagate_kernel()(src, row, col)


R = 1000
G = N // R

def _dot(a, b):
    return jnp.dot(a, b)


def _full(shape):
    nd = len(shape)
    return pl.BlockSpec(shape, lambda i, _nd=nd: (0,) * _nd)


def _rows(shape, axis):
    def idx(i, _axis=axis, _nd=len(shape)):
        return tuple(i if d == _axis else 0 for d in range(_nd))
    return pl.BlockSpec(shape, idx)


def _tc1_body(x_r, degp_r, w1_r, b1_r, w2_r, b2_r, ai_r, ar_r,
              src_r, root_r, dis_r):
    deg = degp_r[0] + degp_r[1]
    disf = jnp.where(deg > 0, lax.rsqrt(jnp.maximum(deg, 1e-12)), 0.0)
    dis = disf[:, 0:1]
    dis_r[...] = dis
    h1 = jnp.maximum(_dot(x_r[...], w1_r[...]) + b1_r[...], 0.0)
    h = jnp.maximum(_dot(h1, w2_r[...]) + b2_r[...], 0.0)
    hs = dis * h
    for k in range(4):
        src_r[k] = _dot(hs, ai_r[k])
        root_r[k] = _dot(h, ar_r[k])


def _tc1(x, degp, W1, b1, W2, b2, A1i, A1r):
    return pl.pallas_call(
        _tc1_body,
        grid=(G,),
        in_specs=[
            _rows((R, 4), 0),
            _rows((NC, R, F), 1),
            _full((4, 32)), _full((1, 32)),
            _full((32, 128)), _full((1, 128)),
            _full((4, 128, 32)), _full((4, 128, 32)),
        ],
        out_specs=[
            _rows((4, R, 32), 1),
            _rows((4, R, 32), 1),
            _rows((R, 1), 0),
        ],
        out_shape=[
            jax.ShapeDtypeStruct((4, N, 32), jnp.float32),
            jax.ShapeDtypeStruct((4, N, 32), jnp.float32),
            jax.ShapeDtypeStruct((N, 1), jnp.float32),
        ],
    )(x, degp, W1, b1, W2, b2, A1i, A1r)


def _tc2_body(p0_r, p1_r, p2_r, p3_r, root_r, bias_r, dis_r, h_r, hs_r):
    dis = dis_r[...]
    acc = jnp.zeros((R, 32), jnp.float32)
    for k, p_r in enumerate((p0_r, p1_r, p2_r, p3_r)):
        agg = dis * (p_r[0] + p_r[1])
        acc = acc + jnp.maximum(agg + root_r[k] + bias_r[k], 0.0)
    h = acc * 0.25
    h_r[...] = h
    hs_r[...] = dis * h


def _tc2(p0, p1, p2, p3, root1, bias1, dis):
    return pl.pallas_call(
        _tc2_body,
        grid=(G,),
        in_specs=[
            _rows((NC, R, 32), 1), _rows((NC, R, 32), 1),
            _rows((NC, R, 32), 1), _rows((NC, R, 32), 1),
            _rows((4, R, 32), 1),
            _full((4, 32)),
            _rows((R, 1), 0),
        ],
        out_specs=[_rows((R, 32), 0), _rows((R, 32), 0)],
        out_shape=[
            jax.ShapeDtypeStruct((N, 32), jnp.float32),
            jax.ShapeDtypeStruct((N, 32), jnp.float32),
        ],
    )(p0, p1, p2, p3, root1, bias1, dis)


def _tc3_body(p_r, hprev_r, dis_r, ai_r, ar_r, bias_r, h_r, hs_r):
    dis = dis_r[...]
    ps = dis * (p_r[0] + p_r[1])
    hp = hprev_r[...]
    acc = jnp.zeros((R, 32), jnp.float32)
    for k in range(4):
        acc = acc + jnp.maximum(
            _dot(ps, ai_r[k]) + _dot(hp, ar_r[k]) + bias_r[k], 0.0)
    h = acc * 0.25
    h_r[...] = h
    hs_r[...] = dis * h


def _tc3(p, hprev, dis, Ai, Ar, bias):
    return pl.pallas_call(
        _tc3_body,
        grid=(G,),
        in_specs=[
            _rows((NC, R, 32), 1),
            _rows((R, 32), 0),
            _rows((R, 1), 0),
            _full((4, 32, 32)), _full((4, 32, 32)), _full((4, 32)),
        ],
        out_specs=[_rows((R, 32), 0), _rows((R, 32), 0)],
        out_shape=[
            jax.ShapeDtypeStruct((N, 32), jnp.float32),
            jax.ShapeDtypeStruct((N, 32), jnp.float32),
        ],
    )(p, hprev, dis, Ai, Ar, bias)


def _tc4_body(h_r, w3_r, b3_r, w4_r, b4_r, out_r):
    t = jnp.maximum(_dot(h_r[...], w3_r[...]) + b3_r[...], 0.0)
    out_r[...] = _dot(t, w4_r[...]) + b4_r[...]


def _tc4(h, W3, b3, W4, b4):
    return pl.pallas_call(
        _tc4_body,
        grid=(G,),
        in_specs=[
            _rows((R, 32), 0),
            _full((32, 16)), _full((1, 16)),
            _full((16, 2)), _full((1, 2)),
        ],
        out_specs=_rows((R, 2), 0),
        out_shape=jax.ShapeDtypeStruct((N, 2), jnp.float32),
    )(h, W3, b3, W4, b4)


def kernel(x, edge_index, W1, b1, W2, b2,
           a1_init, a1_root, a1_bias, a2_init, a2_root, a2_bias,
           a3_init, a3_root, a3_bias, a4_init, a4_root, a4_bias,
           W3, b3, W4, b4):
    row2 = edge_index[0]
    col2 = edge_index[1]

    e_src = jnp.zeros((N, F), jnp.float32).at[:, 0].set(1.0)
    degp = _propagate(e_src, row2, col2)
    src, root1, dis = _tc1(x, degp, W1, b1.reshape(1, 32),
                           W2, b2.reshape(1, 128), a1_init, a1_root)

    p0, p1, p2, p3 = _propagate4_kernel()(
        src[0], src[1], src[2], src[3], row2, col2)
    h, hs = _tc2(p0, p1, p2, p3, root1, a1_bias.reshape(4, 32), dis)

    for Ai, Ar, Ab in ((a2_init, a2_root, a2_bias),
                       (a3_init, a3_root, a3_bias),
                       (a4_init, a4_root, a4_bias)):
        pp = _propagate(hs, row2, col2)
        h, hs = _tc3(pp, h, dis, Ai, Ar, Ab.reshape(4, 32))

    return _tc4(h, W3, b3.reshape(1, 16), W4, b4.reshape(1, 2))
